# Initial kernel scaffold; baseline (speedup 1.0000x reference)
import jax
import jax.numpy as jnp
from jax.experimental import pallas as pl


def kernel(x, pos, batch, W1, b1, W2, b2):
    raise NotImplementedError("write your pallas kernel here")


# speedup vs baseline: 2.9698x; 2.9698x over previous
import functools

import jax
import jax.numpy as jnp
from jax.experimental import pallas as pl
from jax.experimental.pallas import tpu as pltpu

_N = 10000
_M = 2500
_NPAD = 10240
_ROWS = _NPAD // 128
_R2 = 0.2 * 0.2
_MAX_NB = 64


def _fps_kernel(px_ref, py_ref, pz_ref, idx_ref, qx_ref, qy_ref, qz_ref,
                mind_ref):
    lin = (jax.lax.broadcasted_iota(jnp.int32, (_ROWS, 128), 0) * 128
           + jax.lax.broadcasted_iota(jnp.int32, (_ROWS, 128), 1))
    valid = lin < _N
    px = px_ref[...]
    py = py_ref[...]
    pz = pz_ref[...]

    q0x = px_ref[0, 0]
    q0y = py_ref[0, 0]
    q0z = pz_ref[0, 0]
    dx = px - q0x
    dy = py - q0y
    dz = pz - q0z
    d0 = (dx * dx + dy * dy) + dz * dz
    mind_ref[...] = jnp.where(valid, d0, -1.0)
    idx_ref[0] = 0
    qx_ref[0] = q0x
    qy_ref[0] = q0y
    qz_ref[0] = q0z

    def body(i, _):
        mind = mind_ref[...]
        mx = jnp.max(mind)
        nxt = jnp.min(jnp.where(mind == mx, lin, _NPAD))
        sel = lin == nxt
        qx = jnp.sum(jnp.where(sel, px, 0.0))
        qy = jnp.sum(jnp.where(sel, py, 0.0))
        qz = jnp.sum(jnp.where(sel, pz, 0.0))
        ddx = px - qx
        ddy = py - qy
        ddz = pz - qz
        d = (ddx * ddx + ddy * ddy) + ddz * ddz
        mind_ref[...] = jnp.minimum(mind, jnp.where(valid, d, -1.0))
        idx_ref[i] = nxt
        qx_ref[i] = qx
        qy_ref[i] = qy
        qz_ref[i] = qz
        return 0

    jax.lax.fori_loop(1, _M, body, 0)


def _fps(pos):
    coords = jnp.pad(pos, ((0, _NPAD - _N), (0, 0)))
    px = coords[:, 0].reshape(_ROWS, 128)
    py = coords[:, 1].reshape(_ROWS, 128)
    pz = coords[:, 2].reshape(_ROWS, 128)
    out_shape = (
        jax.ShapeDtypeStruct((_M,), jnp.int32),
        jax.ShapeDtypeStruct((_M,), jnp.float32),
        jax.ShapeDtypeStruct((_M,), jnp.float32),
        jax.ShapeDtypeStruct((_M,), jnp.float32),
    )
    idx, qx, qy, qz = pl.pallas_call(
        _fps_kernel,
        out_shape=out_shape,
        out_specs=tuple(pl.BlockSpec(memory_space=pltpu.SMEM)
                        for _ in range(4)),
        scratch_shapes=[pltpu.VMEM((_ROWS, 128), jnp.float32)],
    )(px, py, pz)
    return idx, jnp.stack([qx, qy, qz], axis=1)


def kernel(x, pos, batch, W1, b1, W2, b2):
    idx, pos_q = _fps(pos)
    sq = jnp.sum((pos_q[:, None, :] - pos[None, :, :]) ** 2, axis=-1)
    within = sq < _R2
    neg = jnp.where(within, -sq, -jnp.inf)
    vals, nbr = jax.lax.top_k(neg, _MAX_NB)
    valid = vals > -jnp.inf
    x_j = x[nbr]
    rel = pos[nbr] - pos_q[:, None, :]
    h = jnp.concatenate([x_j, rel], axis=-1)
    h = jax.nn.relu(h @ W1 + b1)
    h = jax.nn.relu(h @ W2 + b2)
    h = jnp.where(valid[:, :, None], h, -jnp.inf)
    out = jnp.max(h, axis=1)
    return (out, pos_q, batch[idx])


# speedup vs baseline: 12.7407x; 4.2900x over previous
import numpy as np

import jax
import jax.numpy as jnp
from jax import lax
from jax.experimental import pallas as pl
from jax.experimental.pallas import tpu as pltpu
from jax.experimental.pallas import tpu_sc as plsc

_N = 10000
_M = 2500
_NPAD = 10240
_ROWS = _NPAD // 128
_R2 = 0.2 * 0.2
_R2F = float(np.float32(_R2))
_R2BITS = int(np.float32(_R2).view(np.int32))
_SENT = int(np.int32(0x7F000000))
_NEG_INF = float("-inf")

_MP = 2560
_NW = 32
_RPW = _MP // _NW
_NCH = _NPAD // 16
_C = 128
_NCHK = _RPW * 64 // _C


def _fps_kernel(px_ref, py_ref, pz_ref, idx_ref, qx_ref, qy_ref, qz_ref,
                mind_ref):
    lin = (jax.lax.broadcasted_iota(jnp.int32, (_ROWS, 128), 0) * 128
           + jax.lax.broadcasted_iota(jnp.int32, (_ROWS, 128), 1))
    valid = lin < _N
    px = px_ref[...]
    py = py_ref[...]
    pz = pz_ref[...]

    q0x = px_ref[0, 0]
    q0y = py_ref[0, 0]
    q0z = pz_ref[0, 0]
    dx = px - q0x
    dy = py - q0y
    dz = pz - q0z
    d0 = (dx * dx + dy * dy) + dz * dz
    mind_ref[...] = jnp.where(valid, d0, -1.0)
    idx_ref[0] = 0
    qx_ref[0] = q0x
    qy_ref[0] = q0y
    qz_ref[0] = q0z

    def body(i, _):
        mind = mind_ref[...]
        mx = jnp.max(mind)
        nxt = jnp.min(jnp.where(mind == mx, lin, _NPAD))
        sel = lin == nxt
        qx = jnp.sum(jnp.where(sel, px, 0.0))
        qy = jnp.sum(jnp.where(sel, py, 0.0))
        qz = jnp.sum(jnp.where(sel, pz, 0.0))
        ddx = px - qx
        ddy = py - qy
        ddz = pz - qz
        d = (ddx * ddx + ddy * ddy) + ddz * ddz
        mind_ref[...] = jnp.minimum(mind, jnp.where(valid, d, -1.0))
        idx_ref[i] = nxt
        qx_ref[i] = qx
        qy_ref[i] = qy
        qz_ref[i] = qz
        return 0

    jax.lax.fori_loop(1, _M, body, 0)


def _fps(pos):
    coords = jnp.pad(pos, ((0, _NPAD - _N), (0, 0)))
    px = coords[:, 0].reshape(_ROWS, 128)
    py = coords[:, 1].reshape(_ROWS, 128)
    pz = coords[:, 2].reshape(_ROWS, 128)
    out_shape = (
        jax.ShapeDtypeStruct((_M,), jnp.int32),
        jax.ShapeDtypeStruct((_M,), jnp.float32),
        jax.ShapeDtypeStruct((_M,), jnp.float32),
        jax.ShapeDtypeStruct((_M,), jnp.float32),
    )
    idx, qx, qy, qz = pl.pallas_call(
        _fps_kernel,
        out_shape=out_shape,
        out_specs=tuple(pl.BlockSpec(memory_space=pltpu.SMEM)
                        for _ in range(4)),
        scratch_shapes=[pltpu.VMEM((_ROWS, 128), jnp.float32)],
    )(px, py, pz)
    return idx, jnp.stack([qx, qy, qz], axis=1)


def _bq_body(pxh, pyh, pzh, pqh, nbrh, vmh, rxh, ryh, rzh,
             pxv, pyv, pzv, pqv, cb, ci, nb, vb, rx, ry, rz):
    wid = lax.axis_index("s") * 2 + lax.axis_index("c")
    base = wid * _RPW
    pltpu.sync_copy(pxh, pxv)
    pltpu.sync_copy(pyh, pyv)
    pltpu.sync_copy(pzh, pzv)
    pltpu.sync_copy(pqh.at[pl.ds(base * 8, _RPW * 8)], pqv)

    i16 = lax.broadcasted_iota(jnp.int32, (16,), 0)
    z16 = jnp.zeros((16,), jnp.int32)
    ones16 = jnp.ones((16,), jnp.int32)
    zf16 = jnp.zeros((16,), jnp.float32)
    ninf16 = jnp.full((16,), _NEG_INF, jnp.float32)

    def row_body(t, _):
        qoff = z16 + t * 8
        qx = plsc.load_gather(pqv, [qoff])
        qy = plsc.load_gather(pqv, [qoff + 1])
        qz = plsc.load_gather(pqv, [qoff + 2])

        def dist_body(c, cnt):
            sl = pl.ds(c * 16, 16)
            dx = pxv[sl] - qx
            dy = pyv[sl] - qy
            dz = pzv[sl] - qz
            d = (dx * dx + dy * dy) + dz * dz
            m = d < _R2F
            db = plsc.bitcast(d, jnp.int32)
            plsc.store_compressed(cb.at[pl.ds(cnt, 16)], db, mask=m)
            plsc.store_compressed(ci.at[pl.ds(cnt, 16)], c * 16 + i16, mask=m)
            return cnt + plsc.all_reduce_population_count(m)[0]

        cnt = lax.fori_loop(0, _NCH, dist_body, jnp.int32(0))
        cb[pl.ds(cnt, 16)] = z16 + _SENT
        nv = (cnt + 15) >> 4

        def bs_body(k, lohi):
            lo, hi = lohi
            mid = (lo + hi) >> 1

            def cnt_body(j, acc):
                b = cb[pl.ds(j * 16, 16)]
                return acc + jnp.where(b <= mid, ones16, z16)

            cle = jnp.sum(lax.fori_loop(0, nv, cnt_body, z16))
            pred = cle >= 64
            return (jnp.where(pred, lo, mid + 1), jnp.where(pred, mid, hi))

        _, thr = lax.fori_loop(0, 30, bs_body,
                               (jnp.int32(0), jnp.int32(_R2BITS)))

        def lt_body(j, acc):
            b = cb[pl.ds(j * 16, 16)]
            return acc + jnp.where(b < thr, ones16, z16)

        cntlt = jnp.sum(lax.fori_loop(0, nv, lt_body, z16))
        quota = 64 - cntlt

        def emit_body(j, carry):
            outc, eqb = carry
            b = cb[pl.ds(j * 16, 16)]
            ii = ci[pl.ds(j * 16, 16)]
            ltm = b < thr
            eqm = b == thr
            eqc = plsc.cumsum(jnp.where(eqm, ones16, z16))
            take = ltm | (eqm & ((eqb + eqc) <= quota))
            plsc.store_compressed(nb.at[pl.ds(t * 64 + outc, 16)], ii,
                                  mask=take)
            outc = outc + plsc.all_reduce_population_count(take)[0]
            eqb = eqb + plsc.all_reduce_population_count(eqm)[0]
            return outc, eqb

        nsel, _ = lax.fori_loop(0, nv, emit_body,
                                (jnp.int32(0), jnp.int32(0)))

        for k in range(4):
            sl = pl.ds(t * 64 + k * 16, 16)
            slot = z16 + k * 16 + i16
            ok = slot < nsel
            idxv = jnp.where(ok, nb[sl], z16)
            nb[sl] = idxv
            vb[sl] = jnp.where(ok, zf16, ninf16)
            rx[sl] = plsc.load_gather(pxv, [idxv]) - qx
            ry[sl] = plsc.load_gather(pyv, [idxv]) - qy
            rz[sl] = plsc.load_gather(pzv, [idxv]) - qz
        return 0

    lax.fori_loop(0, _RPW, row_body, 0)
    sl = pl.ds(base * 64, _RPW * 64)
    pltpu.sync_copy(nb, nbrh.at[sl])
    pltpu.sync_copy(vb, vmh.at[sl])
    pltpu.sync_copy(rx, rxh.at[sl])
    pltpu.sync_copy(ry, ryh.at[sl])
    pltpu.sync_copy(rz, rzh.at[sl])


def _ballquery(px, py, pz, pqflat):
    mesh = plsc.VectorSubcoreMesh(core_axis_name="c", subcore_axis_name="s")
    f = pl.kernel(
        _bq_body,
        compiler_params=pltpu.CompilerParams(needs_layout_passes=False),
        out_type=(
            jax.ShapeDtypeStruct((_MP * 64,), jnp.int32),
            jax.ShapeDtypeStruct((_MP * 64,), jnp.float32),
            jax.ShapeDtypeStruct((_MP * 64,), jnp.float32),
            jax.ShapeDtypeStruct((_MP * 64,), jnp.float32),
            jax.ShapeDtypeStruct((_MP * 64,), jnp.float32),
        ),
        mesh=mesh,
        scratch_types=[
            pltpu.VMEM((_NPAD,), jnp.float32),
            pltpu.VMEM((_NPAD,), jnp.float32),
            pltpu.VMEM((_NPAD,), jnp.float32),
            pltpu.VMEM((_RPW * 8,), jnp.float32),
            pltpu.VMEM((_NPAD + 16,), jnp.int32),
            pltpu.VMEM((_NPAD + 16,), jnp.int32),
            pltpu.VMEM((_RPW * 64,), jnp.int32),
            pltpu.VMEM((_RPW * 64,), jnp.float32),
            pltpu.VMEM((_RPW * 64,), jnp.float32),
            pltpu.VMEM((_RPW * 64,), jnp.float32),
            pltpu.VMEM((_RPW * 64,), jnp.float32),
        ],
    )
    return f(px, py, pz, pqflat)


def _gather_body(xh, nbh, idxh, bh, xgh, bouth,
                 nbv, dstv, bufa, bufb, idxv, bbuf, gsem, ssem):
    wid = lax.axis_index("s") * 2 + lax.axis_index("c")
    base = wid * _RPW
    i16 = lax.broadcasted_iota(jnp.int32, (16,), 0)
    pltpu.sync_copy(nbh.at[pl.ds(base * 64, _RPW * 64)], nbv)

    def dst_body(k, _):
        e = k * 16 + i16
        t = e & 63
        iloc = e >> 6
        dst = t * _MP + base + iloc
        r = k >> 3
        c = (k & 7) * 16
        dstv[r, pl.ds(c, 16)] = dst
        return 0

    lax.fori_loop(0, _RPW * 64 // 16, dst_body, 0)

    pltpu.sync_copy(idxh.at[pl.ds(base, _RPW)], idxv)
    pltpu.async_copy(bh.at[idxv], bbuf, gsem).wait()
    pltpu.sync_copy(bbuf, bouth.at[pl.ds(base, _RPW)])

    bufs = (bufa, bufb)

    def start_g(c):
        return pltpu.async_copy(xh.at[nbv.at[pl.ds(c * _C, _C)]],
                                bufs[c % 2], gsem)

    def start_s(c):
        return pltpu.async_copy(bufs[c % 2], xgh.at[dstv.at[c]], ssem)

    scat = [None] * _NCHK
    gat = [None] * _NCHK
    gat[0] = start_g(0)
    for c in range(_NCHK):
        if c + 1 < _NCHK:
            if c >= 1:
                scat[c - 1].wait()
            gat[c + 1] = start_g(c + 1)
        gat[c].wait()
        scat[c] = start_s(c)
    scat[_NCHK - 2].wait()
    scat[_NCHK - 1].wait()


def _gather(x, nbf, idxp, batch):
    mesh = plsc.VectorSubcoreMesh(core_axis_name="c", subcore_axis_name="s")
    f = pl.kernel(
        _gather_body,
        compiler_params=pltpu.CompilerParams(needs_layout_passes=False),
        out_type=(
            jax.ShapeDtypeStruct((64 * _MP, 128), jnp.float32),
            jax.ShapeDtypeStruct((_MP,), jnp.int32),
        ),
        mesh=mesh,
        scratch_types=[
            pltpu.VMEM((_RPW * 64,), jnp.int32),
            pltpu.VMEM((_NCHK, _C), jnp.int32),
            pltpu.VMEM((_C, 128), jnp.float32),
            pltpu.VMEM((_C, 128), jnp.float32),
            pltpu.VMEM((_RPW,), jnp.int32),
            pltpu.VMEM((_RPW,), jnp.int32),
            pltpu.SemaphoreType.DMA,
            pltpu.SemaphoreType.DMA,
        ],
    )
    return f(x, nbf, idxp, batch)


def _mlp_kernel(xg_ref, rx_ref, ry_ref, rz_ref, vm_ref, w1_ref, w2_ref,
                aux_ref, o_ref):
    w1 = w1_ref[...]
    w2 = w2_ref[...]
    aux = aux_ref[...]
    acc = jnp.full((128, 128), _NEG_INF, jnp.float32)
    for t in range(64):
        xt = xg_ref[t]
        h = jnp.dot(xt, w1, preferred_element_type=jnp.float32)
        h = h + rx_ref[:, t:t + 1] * aux[0:1, :]
        h = h + ry_ref[:, t:t + 1] * aux[1:2, :]
        h = h + rz_ref[:, t:t + 1] * aux[2:3, :]
        h = jnp.maximum(h + aux[3:4, :], 0.0)
        h2 = jnp.dot(h, w2, preferred_element_type=jnp.float32)
        h2 = jnp.maximum(h2 + aux[4:5, :], 0.0)
        acc = jnp.maximum(acc, h2 + vm_ref[:, t:t + 1])
    o_ref[...] = acc


def _mlp(xg, rx2, ry2, rz2, vm2, w1a, w2, aux):
    grid = (_MP // 128,)
    return pl.pallas_call(
        _mlp_kernel,
        grid=grid,
        in_specs=[
            pl.BlockSpec((64, 128, 128), lambda i: (0, i, 0)),
            pl.BlockSpec((128, 64), lambda i: (i, 0)),
            pl.BlockSpec((128, 64), lambda i: (i, 0)),
            pl.BlockSpec((128, 64), lambda i: (i, 0)),
            pl.BlockSpec((128, 64), lambda i: (i, 0)),
            pl.BlockSpec((128, 128), lambda i: (0, 0)),
            pl.BlockSpec((128, 128), lambda i: (0, 0)),
            pl.BlockSpec((8, 128), lambda i: (0, 0)),
        ],
        out_specs=pl.BlockSpec((128, 128), lambda i: (i, 0)),
        out_shape=jax.ShapeDtypeStruct((_MP, 128), jnp.float32),
    )(xg, rx2, ry2, rz2, vm2, w1a, w2, aux)


def kernel(x, pos, batch, W1, b1, W2, b2):
    idx, pos_q = _fps(pos)

    big = jnp.float32(1e9)
    coords = jnp.concatenate(
        [pos, jnp.full((_NPAD - _N, 3), big, jnp.float32)], axis=0)
    px = coords[:, 0]
    py = coords[:, 1]
    pz = coords[:, 2]

    pq8 = jnp.full((_MP, 8), jnp.float32(2e9), jnp.float32)
    pq8 = pq8.at[:_M, 0:3].set(pos_q)
    pqflat = pq8.reshape(-1)

    nbf, vmf, rxf, ryf, rzf = _ballquery(px, py, pz, pqflat)

    idxp = jnp.zeros((_MP,), jnp.int32).at[:_M].set(idx)
    xg, bout = _gather(x, nbf, idxp, batch)

    aux = jnp.zeros((8, 128), jnp.float32)
    aux = aux.at[0:3, :].set(W1[128:131, :])
    aux = aux.at[3, :].set(b1)
    aux = aux.at[4, :].set(b2)

    out = _mlp(xg.reshape(64, _MP, 128),
               rxf.reshape(_MP, 64), ryf.reshape(_MP, 64),
               rzf.reshape(_MP, 64), vmf.reshape(_MP, 64),
               W1[:128, :], W2, aux)

    return (out[:_M], pos_q, bout[:_M])
